# Initial kernel scaffold; baseline (speedup 1.0000x reference)
import jax
import jax.numpy as jnp
from jax.experimental import pallas as pl


def kernel(src_nodes, dst_nodes, timestamps, edge_features, memory, last_update_ts, basis_freq, phase, msg_W, msg_b, gru_W_ih, gru_W_hh, gru_b_ih, gru_b_hh):
    raise NotImplementedError("write your pallas kernel here")


# speedup vs baseline: 3.6957x; 3.6957x over previous
import functools

import jax
import jax.numpy as jnp
from jax import lax
from jax.experimental import pallas as pl
from jax.experimental.pallas import tpu as pltpu
from jax.experimental.pallas import tpu_sc as plsc

N_NODES = 100000
D_MEM = 128
D_TIME = 128
D_EDGE = 16
D_MSG = 128
N_EDGES = 300000

NC = 2
NS = 16
NW = NC * NS

E_PAD = 307200
EPW = E_PAD // NW
CH = 128

FC_W = 16
N_FC = D_MSG // FC_W
ROWS_PER_SUB = 6272
ACC_ROWS = ROWS_PER_SUB * NS
LAST_ROWS = N_NODES - 15 * ROWS_PER_SUB


def _sc_mesh():
    return plsc.VectorSubcoreMesh(
        core_axis_name="c", subcore_axis_name="s", num_cores=NC,
        num_subcores=NS)


def _gather_body(mem_hbm, ts_hbm, src_hbm, dst_hbm,
                 smem_out, dmem_out, sts_out, dts_out,
                 idx_v, rows_v, ts_v, sem):
    wid = lax.axis_index("s") * NC + lax.axis_index("c")
    base = wid * EPW

    def one_table(idx_hbm, mem_out, tsg_out):
        @pl.loop(0, EPW // CH)
        def _(i):
            off = base + i * CH
            pltpu.sync_copy(idx_hbm.at[pl.ds(off, CH)], idx_v)
            pltpu.async_copy(mem_hbm.at[idx_v], rows_v, sem).wait()
            pltpu.sync_copy(rows_v, mem_out.at[pl.ds(off, CH)])
            pltpu.async_copy(ts_hbm.at[idx_v], ts_v, sem).wait()
            pltpu.sync_copy(ts_v, tsg_out.at[pl.ds(off, CH)])

    one_table(src_hbm, smem_out, sts_out)
    one_table(dst_hbm, dmem_out, dts_out)


def _gather(memory, last_ts, src_pad, dst_pad):
    f32 = jnp.float32
    out_type = (
        jax.ShapeDtypeStruct((E_PAD, D_MEM), f32),
        jax.ShapeDtypeStruct((E_PAD, D_MEM), f32),
        jax.ShapeDtypeStruct((E_PAD,), f32),
        jax.ShapeDtypeStruct((E_PAD,), f32),
    )
    scratch = [
        pltpu.VMEM((CH,), jnp.int32),
        pltpu.VMEM((CH, D_MEM), f32),
        pltpu.VMEM((CH,), f32),
        pltpu.SemaphoreType.DMA,
    ]
    return pl.kernel(_gather_body, out_type=out_type, mesh=_sc_mesh(),
                     scratch_types=scratch)(memory, last_ts, src_pad, dst_pad)


BE = 3072


def _msg_body(smem, dmem, sts, dts, ts, ef, wa, wb, wc, wd, b, freq, phase,
              msrc_out, mdst_out):
    i = pl.program_id(0)
    row = jax.lax.broadcasted_iota(jnp.int32, (BE, 1), 0) + i * BE
    valid = row < N_EDGES

    s = smem[...]
    d = dmem[...]
    enc_s = jnp.cos((ts[...] - sts[...])[:, None] * freq[...][None, :]
                    + phase[...][None, :])
    enc_d = jnp.cos((ts[...] - dts[...])[:, None] * freq[...][None, :]
                    + phase[...][None, :])

    dot = functools.partial(jnp.dot, preferred_element_type=jnp.float32)
    s_a = dot(s, wa[...])
    s_b = dot(s, wb[...])
    d_a = dot(d, wa[...])
    d_b = dot(d, wb[...])
    e_d = dot(ef[...], wd[...])
    bias = b[...][None, :]
    mdst = jnp.maximum(s_a + d_b + dot(enc_d, wc[...]) + e_d + bias, 0.0)
    msrc = jnp.maximum(d_a + s_b + dot(enc_s, wc[...]) + e_d + bias, 0.0)
    msrc_out[...] = jnp.where(valid, msrc, 0.0)
    mdst_out[...] = jnp.where(valid, mdst, 0.0)


def _messages(smem, dmem, sts, dts, ts_pad, ef_pad, msg_W, msg_b,
              basis_freq, phase):
    f32 = jnp.float32
    wa = msg_W[:, :D_MEM].T
    wb = msg_W[:, D_MEM:2 * D_MEM].T
    wc = msg_W[:, 2 * D_MEM:2 * D_MEM + D_TIME].T
    wd = msg_W[:, 2 * D_MEM + D_TIME:].T
    grid = (E_PAD // BE,)
    eb = lambda i: (i, 0)
    vb = lambda i: (i,)
    full2 = lambda i: (0, 0)
    full1 = lambda i: (0,)
    in_specs = [
        pl.BlockSpec((BE, D_MEM), eb),
        pl.BlockSpec((BE, D_MEM), eb),
        pl.BlockSpec((BE,), vb),
        pl.BlockSpec((BE,), vb),
        pl.BlockSpec((BE,), vb),
        pl.BlockSpec((BE, D_EDGE), eb),
        pl.BlockSpec((D_MEM, D_MSG), full2),
        pl.BlockSpec((D_MEM, D_MSG), full2),
        pl.BlockSpec((D_TIME, D_MSG), full2),
        pl.BlockSpec((D_EDGE, D_MSG), full2),
        pl.BlockSpec((D_MSG,), full1),
        pl.BlockSpec((D_TIME,), full1),
        pl.BlockSpec((D_TIME,), full1),
    ]
    out_specs = [pl.BlockSpec((BE, D_MSG), eb), pl.BlockSpec((BE, D_MSG), eb)]
    out_shape = [jax.ShapeDtypeStruct((E_PAD, D_MSG), f32),
                 jax.ShapeDtypeStruct((E_PAD, D_MSG), f32)]
    return pl.pallas_call(_msg_body, grid=grid, in_specs=in_specs,
                          out_specs=out_specs, out_shape=out_shape)(
        smem, dmem, sts, dts, ts_pad, ef_pad, wa, wb, wc, wd, msg_b,
        basis_freq, phase)


def _scatter_body(sidx_hbm, didx_hbm, msrc_hbm, mdst_hbm, zeros_hbm,
                  sums_out, cnta_out, cntb_out,
                  acc_sh, idx_v, val_v, ones_v, sem):
    c = lax.axis_index("c")
    s = lax.axis_index("s")
    ecount = E_PAD // NS
    ebase = s * ecount

    def rows_split(fn):
        @pl.when(s < NS - 1)
        def _():
            fn(s * ROWS_PER_SUB, ROWS_PER_SUB)

        @pl.when(s == NS - 1)
        def _():
            fn((NS - 1) * ROWS_PER_SUB, LAST_ROWS)

    def zero_acc():
        def z(r0, nr):
            r0 = pl.multiple_of(r0, 8)
            pltpu.sync_copy(zeros_hbm.at[pl.ds(r0, nr)],
                            acc_sh.at[pl.ds(r0, nr)])
        rows_split(z)

    def scan_array(idx_hbm, val_hbm, fc):
        @pl.loop(0, ecount // CH)
        def _(i):
            off = ebase + i * CH
            pltpu.sync_copy(idx_hbm.at[pl.ds(off, CH)], idx_v)
            pltpu.sync_copy(val_hbm.at[pl.ds(off, CH), pl.ds(fc * FC_W, FC_W)],
                            val_v)
            pltpu.async_copy(val_v, acc_sh.at[idx_v], sem, add=True).wait()

    def drain(out_ref, fc):
        def d(r0, nr):
            r0 = pl.multiple_of(r0, 8)
            pltpu.sync_copy(
                acc_sh.at[pl.ds(r0, nr)],
                out_ref.at[pl.ds(r0, nr), pl.ds(fc * FC_W, FC_W)])
        rows_split(d)

    for fc in range(N_FC):
        @pl.when(c == fc // (N_FC // NC))
        def _(fc=fc):
            zero_acc()
            plsc.subcore_barrier()
            scan_array(sidx_hbm, msrc_hbm, fc)
            scan_array(didx_hbm, mdst_hbm, fc)
            plsc.subcore_barrier()
            drain(sums_out, fc)
            plsc.subcore_barrier()

    def fill_ones(valid):
        @pl.loop(0, CH)
        def _(r):
            ones_v[r, :] = (jnp.where(r < valid, 1.0, 0.0)
                            * jnp.ones((FC_W,), jnp.float32))

    full_chunks = (N_EDGES - (NS - 1) * (E_PAD // NS)) // CH
    mixed_valid = N_EDGES - (NS - 1) * (E_PAD // NS) - full_chunks * CH

    def count_scan(idx_hbm):
        def one_chunk(off):
            pltpu.sync_copy(idx_hbm.at[pl.ds(off, CH)], idx_v)
            pltpu.async_copy(ones_v, acc_sh.at[idx_v], sem, add=True).wait()

        @pl.when(s < NS - 1)
        def _():
            @pl.loop(0, ecount // CH)
            def _(i):
                one_chunk(ebase + i * CH)

        @pl.when(s == NS - 1)
        def _():
            @pl.loop(0, full_chunks)
            def _(i):
                one_chunk(ebase + i * CH)
            fill_ones(mixed_valid)
            one_chunk(ebase + full_chunks * CH)

    for side in range(NC):
        @pl.when(c == side)
        def _(side=side):
            zero_acc()
            fill_ones(CH)
            plsc.subcore_barrier()
            count_scan((sidx_hbm, didx_hbm)[side])
            plsc.subcore_barrier()
            out_ref = (cnta_out, cntb_out)[side]

            def dc(r0, nr):
                r0 = pl.multiple_of(r0, 8)
                pltpu.sync_copy(acc_sh.at[pl.ds(r0, nr)],
                                out_ref.at[pl.ds(r0, nr), pl.ds(0, FC_W)])
            rows_split(dc)


def _scatter(src_pad, dst_pad, msrc, mdst, zeros2d):
    f32 = jnp.float32
    out_type = (
        jax.ShapeDtypeStruct((N_NODES, D_MSG), f32),
        jax.ShapeDtypeStruct((ACC_ROWS, 128), f32),
        jax.ShapeDtypeStruct((ACC_ROWS, 128), f32),
    )
    scratch = [
        pltpu.VMEM_SHARED((ACC_ROWS, FC_W), f32),
        pltpu.VMEM((CH,), jnp.int32),
        pltpu.VMEM((CH, FC_W), f32),
        pltpu.VMEM((CH, FC_W), f32),
        pltpu.SemaphoreType.DMA,
    ]
    return pl.kernel(_scatter_body, out_type=out_type, mesh=_sc_mesh(),
                     scratch_types=scratch,
                     compiler_params=pltpu.CompilerParams(
                         use_tc_tiling_on_sc=False))(
        src_pad, dst_pad, msrc, mdst, zeros2d)


NB = 2000


def _gru_body(sums, cnta, cntb, mem, wih, whh, bih, bhh, out):
    cnt = cnta[...][:, :1] + cntb[...][:, :1]
    agg = sums[...] / jnp.maximum(cnt, 1.0)
    h = mem[...]
    gi = jnp.dot(agg, wih[...], preferred_element_type=jnp.float32) \
        + bih[...][None, :]
    gh = jnp.dot(h, whh[...], preferred_element_type=jnp.float32) \
        + bhh[...][None, :]
    r = jax.nn.sigmoid(gi[:, :D_MEM] + gh[:, :D_MEM])
    z = jax.nn.sigmoid(gi[:, D_MEM:2 * D_MEM] + gh[:, D_MEM:2 * D_MEM])
    n = jnp.tanh(gi[:, 2 * D_MEM:] + r * gh[:, 2 * D_MEM:])
    hn = (1.0 - z) * n + z * h
    out[...] = jnp.where(cnt > 0.0, hn, h)


def _gru(sums, cnta, cntb, memory, gru_W_ih, gru_W_hh, gru_b_ih, gru_b_hh):
    f32 = jnp.float32
    wih = gru_W_ih.T
    whh = gru_W_hh.T
    grid = (N_NODES // NB,)
    nb = lambda i: (i, 0)
    full2 = lambda i: (0, 0)
    full1 = lambda i: (0,)
    in_specs = [
        pl.BlockSpec((NB, D_MSG), nb),
        pl.BlockSpec((NB, 128), nb),
        pl.BlockSpec((NB, 128), nb),
        pl.BlockSpec((NB, D_MEM), nb),
        pl.BlockSpec((D_MSG, 3 * D_MEM), full2),
        pl.BlockSpec((D_MEM, 3 * D_MEM), full2),
        pl.BlockSpec((3 * D_MEM,), full1),
        pl.BlockSpec((3 * D_MEM,), full1),
    ]
    out_specs = pl.BlockSpec((NB, D_MEM), nb)
    out_shape = jax.ShapeDtypeStruct((N_NODES, D_MEM), f32)
    return pl.pallas_call(_gru_body, grid=grid, in_specs=in_specs,
                          out_specs=out_specs, out_shape=out_shape)(
        sums, cnta, cntb, memory, wih, whh, gru_b_ih, gru_b_hh)


def kernel(src_nodes, dst_nodes, timestamps, edge_features, memory,
           last_update_ts, basis_freq, phase, msg_W, msg_b,
           gru_W_ih, gru_W_hh, gru_b_ih, gru_b_hh):
    f32 = jnp.float32
    npad = E_PAD - N_EDGES
    pad_idx = (jnp.arange(npad, dtype=jnp.int32) * 1009) % N_NODES
    src_pad = jnp.concatenate([src_nodes.astype(jnp.int32), pad_idx])
    dst_pad = jnp.concatenate([dst_nodes.astype(jnp.int32), pad_idx])
    ts_pad = jnp.concatenate([timestamps, jnp.zeros((npad,), f32)])
    ef_pad = jnp.concatenate(
        [edge_features, jnp.zeros((npad, D_EDGE), f32)], axis=0)
    zeros2d = jnp.zeros((N_NODES, FC_W), f32)

    smem, dmem, sts, dts = _gather(memory, last_update_ts, src_pad, dst_pad)
    msrc, mdst = _messages(smem, dmem, sts, dts, ts_pad, ef_pad, msg_W, msg_b,
                           basis_freq, phase)
    sums, cnta, cntb = _scatter(src_pad, dst_pad, msrc, mdst, zeros2d)
    return _gru(sums, cnta, cntb, memory, gru_W_ih, gru_W_hh,
                gru_b_ih, gru_b_hh)


# speedup vs baseline: 5.3830x; 1.4565x over previous
import functools

import jax
import jax.numpy as jnp
from jax import lax
from jax.experimental import pallas as pl
from jax.experimental.pallas import tpu as pltpu
from jax.experimental.pallas import tpu_sc as plsc

N_NODES = 100000
D_MEM = 128
D_TIME = 128
D_EDGE = 16
D_MSG = 128
N_EDGES = 300000

NC = 2
NS = 16
NW = NC * NS

E_PAD = 307200
EPW = E_PAD // NW
CH = 128
KB = 10

FC_W = 16
N_FC = D_MSG // FC_W
ROWS_PER_SUB = 6272
ACC_ROWS = ROWS_PER_SUB * NS
LAST_ROWS = N_NODES - 15 * ROWS_PER_SUB


def _sc_mesh():
    return plsc.VectorSubcoreMesh(
        core_axis_name="c", subcore_axis_name="s", num_cores=NC,
        num_subcores=NS)


def _gather_body(mem_hbm, ts_hbm, src_hbm, dst_hbm,
                 smem_out, dmem_out, sts_out, dts_out,
                 idx_v, rows_v, ts_v, sem):
    wid = lax.axis_index("s") * NC + lax.axis_index("c")
    base = wid * EPW

    def one_table(idx_hbm, mem_out, tsg_out):
        @pl.loop(0, EPW // CH)
        def _(i):
            off = base + i * CH
            pltpu.sync_copy(idx_hbm.at[pl.ds(off, CH)], idx_v)
            pltpu.async_copy(mem_hbm.at[idx_v], rows_v, sem).wait()
            pltpu.sync_copy(rows_v, mem_out.at[pl.ds(off, CH)])
            pltpu.async_copy(ts_hbm.at[idx_v], ts_v, sem).wait()
            pltpu.sync_copy(ts_v, tsg_out.at[pl.ds(off, CH)])

    one_table(src_hbm, smem_out, sts_out)
    one_table(dst_hbm, dmem_out, dts_out)


def _gather(memory, last_ts, src_pad, dst_pad):
    f32 = jnp.float32
    out_type = (
        jax.ShapeDtypeStruct((E_PAD, D_MEM), f32),
        jax.ShapeDtypeStruct((E_PAD, D_MEM), f32),
        jax.ShapeDtypeStruct((E_PAD,), f32),
        jax.ShapeDtypeStruct((E_PAD,), f32),
    )
    scratch = [
        pltpu.VMEM((CH,), jnp.int32),
        pltpu.VMEM((CH, D_MEM), f32),
        pltpu.VMEM((CH,), f32),
        pltpu.SemaphoreType.DMA,
    ]
    return pl.kernel(_gather_body, out_type=out_type, mesh=_sc_mesh(),
                     scratch_types=scratch)(memory, last_ts, src_pad, dst_pad)


BE = 3072


def _msg_body(smem, dmem, sts, dts, ts, ef, wa, wb, wc, wd, b, freq, phase,
              msrc_out, mdst_out):
    i = pl.program_id(0)
    row = jax.lax.broadcasted_iota(jnp.int32, (BE, 1), 0) + i * BE
    valid = row < N_EDGES

    s = smem[...]
    d = dmem[...]
    enc_s = jnp.cos((ts[...] - sts[...])[:, None] * freq[...][None, :]
                    + phase[...][None, :])
    enc_d = jnp.cos((ts[...] - dts[...])[:, None] * freq[...][None, :]
                    + phase[...][None, :])

    dot = functools.partial(jnp.dot, preferred_element_type=jnp.float32)
    s_a = dot(s, wa[...])
    s_b = dot(s, wb[...])
    d_a = dot(d, wa[...])
    d_b = dot(d, wb[...])
    e_d = dot(ef[...], wd[...])
    bias = b[...][None, :]
    mdst = jnp.maximum(s_a + d_b + dot(enc_d, wc[...]) + e_d + bias, 0.0)
    msrc = jnp.maximum(d_a + s_b + dot(enc_s, wc[...]) + e_d + bias, 0.0)
    msrc_out[...] = jnp.where(valid, msrc, 0.0)
    mdst_out[...] = jnp.where(valid, mdst, 0.0)


def _messages(smem, dmem, sts, dts, ts_pad, ef_pad, msg_W, msg_b,
              basis_freq, phase):
    f32 = jnp.float32
    wa = msg_W[:, :D_MEM].T
    wb = msg_W[:, D_MEM:2 * D_MEM].T
    wc = msg_W[:, 2 * D_MEM:2 * D_MEM + D_TIME].T
    wd = msg_W[:, 2 * D_MEM + D_TIME:].T
    grid = (E_PAD // BE,)
    eb = lambda i: (i, 0)
    vb = lambda i: (i,)
    full2 = lambda i: (0, 0)
    full1 = lambda i: (0,)
    in_specs = [
        pl.BlockSpec((BE, D_MEM), eb),
        pl.BlockSpec((BE, D_MEM), eb),
        pl.BlockSpec((BE,), vb),
        pl.BlockSpec((BE,), vb),
        pl.BlockSpec((BE,), vb),
        pl.BlockSpec((BE, D_EDGE), eb),
        pl.BlockSpec((D_MEM, D_MSG), full2),
        pl.BlockSpec((D_MEM, D_MSG), full2),
        pl.BlockSpec((D_TIME, D_MSG), full2),
        pl.BlockSpec((D_EDGE, D_MSG), full2),
        pl.BlockSpec((D_MSG,), full1),
        pl.BlockSpec((D_TIME,), full1),
        pl.BlockSpec((D_TIME,), full1),
    ]
    out_specs = [pl.BlockSpec((BE, D_MSG), eb), pl.BlockSpec((BE, D_MSG), eb)]
    out_shape = [jax.ShapeDtypeStruct((E_PAD, D_MSG), f32),
                 jax.ShapeDtypeStruct((E_PAD, D_MSG), f32)]
    return pl.pallas_call(_msg_body, grid=grid, in_specs=in_specs,
                          out_specs=out_specs, out_shape=out_shape)(
        smem, dmem, sts, dts, ts_pad, ef_pad, wa, wb, wc, wd, msg_b,
        basis_freq, phase)


def _scatter_body(sidx2_hbm, didx2_hbm, msrc_hbm, mdst_hbm, zeros_hbm,
                  sums_out, cnta_out, cntb_out,
                  acc_sh, idxb_v, valb_v, ones_v, sem):
    c = lax.axis_index("c")
    s = lax.axis_index("s")
    ecount = E_PAD // NS
    ebase = s * ecount

    def rows_split(fn):
        @pl.when(s < NS - 1)
        def _():
            fn(s * ROWS_PER_SUB, ROWS_PER_SUB)

        @pl.when(s == NS - 1)
        def _():
            fn((NS - 1) * ROWS_PER_SUB, LAST_ROWS)

    def zero_acc():
        def z(r0, nr):
            r0 = pl.multiple_of(r0, 8)
            pltpu.sync_copy(zeros_hbm.at[pl.ds(r0, nr)],
                            acc_sh.at[pl.ds(r0, nr)])
        rows_split(z)

    def scan_array(idx2_hbm, val_hbm, fc):
        nchunk = ecount // CH
        crow = s * nchunk

        @pl.loop(0, nchunk // KB)
        def _(b):
            row0 = crow + b * KB
            e_off = row0 * CH
            cp1 = pltpu.async_copy(idx2_hbm.at[pl.ds(row0, KB)], idxb_v, sem)
            cp2 = pltpu.async_copy(
                val_hbm.at[pl.ds(e_off, KB * CH), pl.ds(fc * FC_W, FC_W)],
                valb_v, sem)
            cp1.wait()
            cp2.wait()
            descs = []
            for j in range(KB):
                descs.append(pltpu.async_copy(
                    valb_v.at[pl.ds(j * CH, CH)],
                    acc_sh.at[idxb_v.at[j]], sem, add=True))
            for d in descs:
                d.wait()

    def drain(out_ref, fc):
        def d(r0, nr):
            r0 = pl.multiple_of(r0, 8)
            pltpu.sync_copy(
                acc_sh.at[pl.ds(r0, nr)],
                out_ref.at[pl.ds(r0, nr), pl.ds(fc * FC_W, FC_W)])
        rows_split(d)

    for fc in range(N_FC):
        @pl.when(c == fc // (N_FC // NC))
        def _(fc=fc):
            zero_acc()
            plsc.subcore_barrier()
            scan_array(sidx2_hbm, msrc_hbm, fc)
            scan_array(didx2_hbm, mdst_hbm, fc)
            plsc.subcore_barrier()
            drain(sums_out, fc)
            plsc.subcore_barrier()

    def fill_ones(valid):
        @pl.loop(0, CH)
        def _(r):
            ones_v[r, :] = (jnp.where(r < valid, 1.0, 0.0)
                            * jnp.ones((FC_W,), jnp.float32))

    full_chunks = (N_EDGES - (NS - 1) * (E_PAD // NS)) // CH
    mixed_valid = N_EDGES - (NS - 1) * (E_PAD // NS) - full_chunks * CH

    def count_scan(idx2_hbm):
        crow = s * (ecount // CH)

        def one_chunk(row):
            pltpu.sync_copy(idx2_hbm.at[pl.ds(row, 1)],
                            idxb_v.at[pl.ds(0, 1)])
            pltpu.async_copy(ones_v, acc_sh.at[idxb_v.at[0]],
                             sem, add=True).wait()

        @pl.when(s < NS - 1)
        def _():
            @pl.loop(0, ecount // CH)
            def _(i):
                one_chunk(crow + i)

        @pl.when(s == NS - 1)
        def _():
            @pl.loop(0, full_chunks)
            def _(i):
                one_chunk(crow + i)
            fill_ones(mixed_valid)
            one_chunk(crow + full_chunks)

    for side in range(NC):
        @pl.when(c == side)
        def _(side=side):
            zero_acc()
            fill_ones(CH)
            plsc.subcore_barrier()
            count_scan((sidx2_hbm, didx2_hbm)[side])
            plsc.subcore_barrier()
            out_ref = (cnta_out, cntb_out)[side]

            def dc(r0, nr):
                r0 = pl.multiple_of(r0, 8)
                pltpu.sync_copy(acc_sh.at[pl.ds(r0, nr)],
                                out_ref.at[pl.ds(r0, nr), pl.ds(0, FC_W)])
            rows_split(dc)


def _scatter(src_pad, dst_pad, msrc, mdst, zeros2d):
    f32 = jnp.float32
    out_type = (
        jax.ShapeDtypeStruct((N_NODES, D_MSG), f32),
        jax.ShapeDtypeStruct((ACC_ROWS, 128), f32),
        jax.ShapeDtypeStruct((ACC_ROWS, 128), f32),
    )
    scratch = [
        pltpu.VMEM_SHARED((ACC_ROWS, FC_W), f32),
        pltpu.VMEM((KB, CH), jnp.int32),
        pltpu.VMEM((KB * CH, FC_W), f32),
        pltpu.VMEM((CH, FC_W), f32),
        pltpu.SemaphoreType.DMA,
    ]
    return pl.kernel(_scatter_body, out_type=out_type, mesh=_sc_mesh(),
                     scratch_types=scratch,
                     compiler_params=pltpu.CompilerParams(
                         use_tc_tiling_on_sc=False))(
        src_pad.reshape(-1, CH), dst_pad.reshape(-1, CH), msrc, mdst, zeros2d)


NB = 2000


def _gru_body(sums, cnta, cntb, mem, wih, whh, bih, bhh, out):
    cnt = cnta[...][:, :1] + cntb[...][:, :1]
    agg = sums[...] / jnp.maximum(cnt, 1.0)
    h = mem[...]
    gi = jnp.dot(agg, wih[...], preferred_element_type=jnp.float32) \
        + bih[...][None, :]
    gh = jnp.dot(h, whh[...], preferred_element_type=jnp.float32) \
        + bhh[...][None, :]
    r = jax.nn.sigmoid(gi[:, :D_MEM] + gh[:, :D_MEM])
    z = jax.nn.sigmoid(gi[:, D_MEM:2 * D_MEM] + gh[:, D_MEM:2 * D_MEM])
    n = jnp.tanh(gi[:, 2 * D_MEM:] + r * gh[:, 2 * D_MEM:])
    hn = (1.0 - z) * n + z * h
    out[...] = jnp.where(cnt > 0.0, hn, h)


def _gru(sums, cnta, cntb, memory, gru_W_ih, gru_W_hh, gru_b_ih, gru_b_hh):
    f32 = jnp.float32
    wih = gru_W_ih.T
    whh = gru_W_hh.T
    grid = (N_NODES // NB,)
    nb = lambda i: (i, 0)
    full2 = lambda i: (0, 0)
    full1 = lambda i: (0,)
    in_specs = [
        pl.BlockSpec((NB, D_MSG), nb),
        pl.BlockSpec((NB, 128), nb),
        pl.BlockSpec((NB, 128), nb),
        pl.BlockSpec((NB, D_MEM), nb),
        pl.BlockSpec((D_MSG, 3 * D_MEM), full2),
        pl.BlockSpec((D_MEM, 3 * D_MEM), full2),
        pl.BlockSpec((3 * D_MEM,), full1),
        pl.BlockSpec((3 * D_MEM,), full1),
    ]
    out_specs = pl.BlockSpec((NB, D_MEM), nb)
    out_shape = jax.ShapeDtypeStruct((N_NODES, D_MEM), f32)
    return pl.pallas_call(_gru_body, grid=grid, in_specs=in_specs,
                          out_specs=out_specs, out_shape=out_shape)(
        sums, cnta, cntb, memory, wih, whh, gru_b_ih, gru_b_hh)


def kernel(src_nodes, dst_nodes, timestamps, edge_features, memory,
           last_update_ts, basis_freq, phase, msg_W, msg_b,
           gru_W_ih, gru_W_hh, gru_b_ih, gru_b_hh):
    f32 = jnp.float32
    npad = E_PAD - N_EDGES
    pad_idx = (jnp.arange(npad, dtype=jnp.int32) * 1009) % N_NODES
    src_pad = jnp.concatenate([src_nodes.astype(jnp.int32), pad_idx])
    dst_pad = jnp.concatenate([dst_nodes.astype(jnp.int32), pad_idx])
    ts_pad = jnp.concatenate([timestamps, jnp.zeros((npad,), f32)])
    ef_pad = jnp.concatenate(
        [edge_features, jnp.zeros((npad, D_EDGE), f32)], axis=0)
    zeros2d = jnp.zeros((N_NODES, FC_W), f32)

    smem, dmem, sts, dts = _gather(memory, last_update_ts, src_pad, dst_pad)
    msrc, mdst = _messages(smem, dmem, sts, dts, ts_pad, ef_pad, msg_W, msg_b,
                           basis_freq, phase)
    sums, cnta, cntb = _scatter(src_pad, dst_pad, msrc, mdst, zeros2d)
    return _gru(sums, cnta, cntb, memory, gru_W_ih, gru_W_hh,
                gru_b_ih, gru_b_hh)


# speedup vs baseline: 6.1157x; 1.1361x over previous
import functools

import jax
import jax.numpy as jnp
from jax import lax
from jax.experimental import pallas as pl
from jax.experimental.pallas import tpu as pltpu
from jax.experimental.pallas import tpu_sc as plsc

N_NODES = 100000
D_MEM = 128
D_TIME = 128
D_EDGE = 16
D_MSG = 128
N_EDGES = 300000

NC = 2
NS = 16
NW = NC * NS

E_PAD = 307200
EPW = E_PAD // NW
CH = 128
KB = 10

FC_W = 16
N_FC = D_MSG // FC_W
ROWS_PER_SUB = 6272
ACC_ROWS = ROWS_PER_SUB * NS
LAST_ROWS = N_NODES - 15 * ROWS_PER_SUB


def _sc_mesh():
    return plsc.VectorSubcoreMesh(
        core_axis_name="c", subcore_axis_name="s", num_cores=NC,
        num_subcores=NS)


KG = 5


def _gather_body(mem_hbm, ts_hbm, src2_hbm, dst2_hbm,
                 smem_out, dmem_out, sts_out, dts_out,
                 idx_v, rows_v, ts_v, sem):
    wid = lax.axis_index("s") * NC + lax.axis_index("c")
    crow0 = wid * (EPW // CH)

    def one_table(idx2_hbm, mem_out, tsg_out):
        @pl.loop(0, EPW // CH // KG)
        def _(b):
            row0 = crow0 + b * KG
            e_off = row0 * CH
            pltpu.async_copy(idx2_hbm.at[pl.ds(row0, KG)], idx_v, sem).wait()
            descs = []
            for j in range(KG):
                descs.append(pltpu.async_copy(
                    mem_hbm.at[idx_v.at[j]],
                    rows_v.at[pl.ds(j * CH, CH)], sem))
                descs.append(pltpu.async_copy(
                    ts_hbm.at[idx_v.at[j]],
                    ts_v.at[pl.ds(j * CH, CH)], sem))
            for d in descs:
                d.wait()
            pltpu.sync_copy(rows_v, mem_out.at[pl.ds(e_off, KG * CH)])
            pltpu.sync_copy(ts_v, tsg_out.at[pl.ds(e_off, KG * CH)])

    one_table(src2_hbm, smem_out, sts_out)
    one_table(dst2_hbm, dmem_out, dts_out)


def _gather(memory, last_ts, src_pad, dst_pad):
    f32 = jnp.float32
    out_type = (
        jax.ShapeDtypeStruct((E_PAD, D_MEM), f32),
        jax.ShapeDtypeStruct((E_PAD, D_MEM), f32),
        jax.ShapeDtypeStruct((E_PAD,), f32),
        jax.ShapeDtypeStruct((E_PAD,), f32),
    )
    scratch = [
        pltpu.VMEM((KG, CH), jnp.int32),
        pltpu.VMEM((KG * CH, D_MEM), f32),
        pltpu.VMEM((KG * CH,), f32),
        pltpu.SemaphoreType.DMA,
    ]
    return pl.kernel(_gather_body, out_type=out_type, mesh=_sc_mesh(),
                     scratch_types=scratch,
                     compiler_params=pltpu.CompilerParams(
                         use_tc_tiling_on_sc=False))(
        memory, last_ts, src_pad.reshape(-1, CH), dst_pad.reshape(-1, CH))


BE = 3072


def _msg_body(smem, dmem, sts, dts, ts, ef, wa, wb, wc, wd, b, freq, phase,
              msrc_out, mdst_out):
    i = pl.program_id(0)
    row = jax.lax.broadcasted_iota(jnp.int32, (BE, 1), 0) + i * BE
    valid = row < N_EDGES

    s = smem[...]
    d = dmem[...]
    enc_s = jnp.cos((ts[...] - sts[...])[:, None] * freq[...][None, :]
                    + phase[...][None, :])
    enc_d = jnp.cos((ts[...] - dts[...])[:, None] * freq[...][None, :]
                    + phase[...][None, :])

    dot = functools.partial(jnp.dot, preferred_element_type=jnp.float32)
    s_a = dot(s, wa[...])
    s_b = dot(s, wb[...])
    d_a = dot(d, wa[...])
    d_b = dot(d, wb[...])
    e_d = dot(ef[...], wd[...])
    bias = b[...][None, :]
    mdst = jnp.maximum(s_a + d_b + dot(enc_d, wc[...]) + e_d + bias, 0.0)
    msrc = jnp.maximum(d_a + s_b + dot(enc_s, wc[...]) + e_d + bias, 0.0)
    msrc_out[...] = jnp.where(valid, msrc, 0.0)
    mdst_out[...] = jnp.where(valid, mdst, 0.0)


def _messages(smem, dmem, sts, dts, ts_pad, ef_pad, msg_W, msg_b,
              basis_freq, phase):
    f32 = jnp.float32
    wa = msg_W[:, :D_MEM].T
    wb = msg_W[:, D_MEM:2 * D_MEM].T
    wc = msg_W[:, 2 * D_MEM:2 * D_MEM + D_TIME].T
    wd = msg_W[:, 2 * D_MEM + D_TIME:].T
    grid = (E_PAD // BE,)
    eb = lambda i: (i, 0)
    vb = lambda i: (i,)
    full2 = lambda i: (0, 0)
    full1 = lambda i: (0,)
    in_specs = [
        pl.BlockSpec((BE, D_MEM), eb),
        pl.BlockSpec((BE, D_MEM), eb),
        pl.BlockSpec((BE,), vb),
        pl.BlockSpec((BE,), vb),
        pl.BlockSpec((BE,), vb),
        pl.BlockSpec((BE, D_EDGE), eb),
        pl.BlockSpec((D_MEM, D_MSG), full2),
        pl.BlockSpec((D_MEM, D_MSG), full2),
        pl.BlockSpec((D_TIME, D_MSG), full2),
        pl.BlockSpec((D_EDGE, D_MSG), full2),
        pl.BlockSpec((D_MSG,), full1),
        pl.BlockSpec((D_TIME,), full1),
        pl.BlockSpec((D_TIME,), full1),
    ]
    out_specs = [pl.BlockSpec((BE, D_MSG), eb), pl.BlockSpec((BE, D_MSG), eb)]
    out_shape = [jax.ShapeDtypeStruct((E_PAD, D_MSG), f32),
                 jax.ShapeDtypeStruct((E_PAD, D_MSG), f32)]
    return pl.pallas_call(_msg_body, grid=grid, in_specs=in_specs,
                          out_specs=out_specs, out_shape=out_shape)(
        smem, dmem, sts, dts, ts_pad, ef_pad, wa, wb, wc, wd, msg_b,
        basis_freq, phase)


def _scatter_body(sidx2_hbm, didx2_hbm, msrc_hbm, mdst_hbm, zeros_hbm,
                  sums_out, cnta_out, cntb_out,
                  acc_sh, idxb_v, valb_v, ones_v, sem):
    c = lax.axis_index("c")
    s = lax.axis_index("s")
    ecount = E_PAD // NS
    ebase = s * ecount

    def rows_split(fn):
        @pl.when(s < NS - 1)
        def _():
            fn(s * ROWS_PER_SUB, ROWS_PER_SUB)

        @pl.when(s == NS - 1)
        def _():
            fn((NS - 1) * ROWS_PER_SUB, LAST_ROWS)

    def zero_acc():
        def z(r0, nr):
            r0 = pl.multiple_of(r0, 8)
            pltpu.sync_copy(zeros_hbm.at[pl.ds(r0, nr)],
                            acc_sh.at[pl.ds(r0, nr)])
        rows_split(z)

    def scan_array(idx2_hbm, val_hbm, fc):
        nchunk = ecount // CH
        crow = s * nchunk

        @pl.loop(0, nchunk // KB)
        def _(b):
            row0 = crow + b * KB
            e_off = row0 * CH
            cp1 = pltpu.async_copy(idx2_hbm.at[pl.ds(row0, KB)], idxb_v, sem)
            cp2 = pltpu.async_copy(
                val_hbm.at[pl.ds(e_off, KB * CH), pl.ds(fc * FC_W, FC_W)],
                valb_v, sem)
            cp1.wait()
            cp2.wait()
            descs = []
            for j in range(KB):
                descs.append(pltpu.async_copy(
                    valb_v.at[pl.ds(j * CH, CH)],
                    acc_sh.at[idxb_v.at[j]], sem, add=True))
            for d in descs:
                d.wait()

    def drain(out_ref, fc):
        def d(r0, nr):
            r0 = pl.multiple_of(r0, 8)
            pltpu.sync_copy(
                acc_sh.at[pl.ds(r0, nr)],
                out_ref.at[pl.ds(r0, nr), pl.ds(fc * FC_W, FC_W)])
        rows_split(d)

    for fc in range(N_FC):
        @pl.when(c == fc // (N_FC // NC))
        def _(fc=fc):
            zero_acc()
            plsc.subcore_barrier()
            scan_array(sidx2_hbm, msrc_hbm, fc)
            scan_array(didx2_hbm, mdst_hbm, fc)
            plsc.subcore_barrier()
            drain(sums_out, fc)
            plsc.subcore_barrier()

    def fill_ones(valid):
        @pl.loop(0, CH)
        def _(r):
            ones_v[r, :] = (jnp.where(r < valid, 1.0, 0.0)
                            * jnp.ones((FC_W,), jnp.float32))

    full_chunks = (N_EDGES - (NS - 1) * (E_PAD // NS)) // CH
    mixed_valid = N_EDGES - (NS - 1) * (E_PAD // NS) - full_chunks * CH

    def count_scan(idx2_hbm):
        crow = s * (ecount // CH)

        def count_batch(row0, n):
            pltpu.async_copy(idx2_hbm.at[pl.ds(row0, n)],
                             idxb_v.at[pl.ds(0, n)], sem).wait()
            descs = [pltpu.async_copy(ones_v, acc_sh.at[idxb_v.at[j]],
                                      sem, add=True) for j in range(n)]
            for d in descs:
                d.wait()

        @pl.when(s < NS - 1)
        def _():
            @pl.loop(0, ecount // CH // KB)
            def _(i):
                count_batch(crow + i * KB, KB)

        @pl.when(s == NS - 1)
        def _():
            nfull_b = full_chunks // KB
            rem = full_chunks - nfull_b * KB

            @pl.loop(0, nfull_b)
            def _(i):
                count_batch(crow + i * KB, KB)
            count_batch(crow + nfull_b * KB, rem)
            fill_ones(mixed_valid)
            count_batch(crow + full_chunks, 1)

    for side in range(NC):
        @pl.when(c == side)
        def _(side=side):
            zero_acc()
            fill_ones(CH)
            plsc.subcore_barrier()
            count_scan((sidx2_hbm, didx2_hbm)[side])
            plsc.subcore_barrier()
            out_ref = (cnta_out, cntb_out)[side]

            def dc(r0, nr):
                r0 = pl.multiple_of(r0, 8)
                pltpu.sync_copy(acc_sh.at[pl.ds(r0, nr)],
                                out_ref.at[pl.ds(r0, nr), pl.ds(0, FC_W)])
            rows_split(dc)


def _scatter(src_pad, dst_pad, msrc, mdst, zeros2d):
    f32 = jnp.float32
    out_type = (
        jax.ShapeDtypeStruct((N_NODES, D_MSG), f32),
        jax.ShapeDtypeStruct((ACC_ROWS, 128), f32),
        jax.ShapeDtypeStruct((ACC_ROWS, 128), f32),
    )
    scratch = [
        pltpu.VMEM_SHARED((ACC_ROWS, FC_W), f32),
        pltpu.VMEM((KB, CH), jnp.int32),
        pltpu.VMEM((KB * CH, FC_W), f32),
        pltpu.VMEM((CH, FC_W), f32),
        pltpu.SemaphoreType.DMA,
    ]
    return pl.kernel(_scatter_body, out_type=out_type, mesh=_sc_mesh(),
                     scratch_types=scratch,
                     compiler_params=pltpu.CompilerParams(
                         use_tc_tiling_on_sc=False))(
        src_pad.reshape(-1, CH), dst_pad.reshape(-1, CH), msrc, mdst, zeros2d)


NB = 2000


def _gru_body(sums, cnta, cntb, mem, wih, whh, bih, bhh, out):
    cnt = cnta[...][:, :1] + cntb[...][:, :1]
    agg = sums[...] / jnp.maximum(cnt, 1.0)
    h = mem[...]
    gi = jnp.dot(agg, wih[...], preferred_element_type=jnp.float32) \
        + bih[...][None, :]
    gh = jnp.dot(h, whh[...], preferred_element_type=jnp.float32) \
        + bhh[...][None, :]
    r = jax.nn.sigmoid(gi[:, :D_MEM] + gh[:, :D_MEM])
    z = jax.nn.sigmoid(gi[:, D_MEM:2 * D_MEM] + gh[:, D_MEM:2 * D_MEM])
    n = jnp.tanh(gi[:, 2 * D_MEM:] + r * gh[:, 2 * D_MEM:])
    hn = (1.0 - z) * n + z * h
    out[...] = jnp.where(cnt > 0.0, hn, h)


def _gru(sums, cnta, cntb, memory, gru_W_ih, gru_W_hh, gru_b_ih, gru_b_hh):
    f32 = jnp.float32
    wih = gru_W_ih.T
    whh = gru_W_hh.T
    grid = (N_NODES // NB,)
    nb = lambda i: (i, 0)
    full2 = lambda i: (0, 0)
    full1 = lambda i: (0,)
    in_specs = [
        pl.BlockSpec((NB, D_MSG), nb),
        pl.BlockSpec((NB, 128), nb),
        pl.BlockSpec((NB, 128), nb),
        pl.BlockSpec((NB, D_MEM), nb),
        pl.BlockSpec((D_MSG, 3 * D_MEM), full2),
        pl.BlockSpec((D_MEM, 3 * D_MEM), full2),
        pl.BlockSpec((3 * D_MEM,), full1),
        pl.BlockSpec((3 * D_MEM,), full1),
    ]
    out_specs = pl.BlockSpec((NB, D_MEM), nb)
    out_shape = jax.ShapeDtypeStruct((N_NODES, D_MEM), f32)
    return pl.pallas_call(_gru_body, grid=grid, in_specs=in_specs,
                          out_specs=out_specs, out_shape=out_shape)(
        sums, cnta, cntb, memory, wih, whh, gru_b_ih, gru_b_hh)


def kernel(src_nodes, dst_nodes, timestamps, edge_features, memory,
           last_update_ts, basis_freq, phase, msg_W, msg_b,
           gru_W_ih, gru_W_hh, gru_b_ih, gru_b_hh):
    f32 = jnp.float32
    npad = E_PAD - N_EDGES
    pad_idx = (jnp.arange(npad, dtype=jnp.int32) * 1009) % N_NODES
    src_pad = jnp.concatenate([src_nodes.astype(jnp.int32), pad_idx])
    dst_pad = jnp.concatenate([dst_nodes.astype(jnp.int32), pad_idx])
    ts_pad = jnp.concatenate([timestamps, jnp.zeros((npad,), f32)])
    ef_pad = jnp.concatenate(
        [edge_features, jnp.zeros((npad, D_EDGE), f32)], axis=0)
    zeros2d = jnp.zeros((N_NODES, FC_W), f32)

    smem, dmem, sts, dts = _gather(memory, last_update_ts, src_pad, dst_pad)
    msrc, mdst = _messages(smem, dmem, sts, dts, ts_pad, ef_pad, msg_W, msg_b,
                           basis_freq, phase)
    sums, cnta, cntb = _scatter(src_pad, dst_pad, msrc, mdst, zeros2d)
    return _gru(sums, cnta, cntb, memory, gru_W_ih, gru_W_hh,
                gru_b_ih, gru_b_hh)


# speedup vs baseline: 6.5205x; 1.0662x over previous
import functools

import jax
import jax.numpy as jnp
from jax import lax
from jax.experimental import pallas as pl
from jax.experimental.pallas import tpu as pltpu
from jax.experimental.pallas import tpu_sc as plsc

N_NODES = 100000
D_MEM = 128
D_TIME = 128
D_EDGE = 16
D_MSG = 128
N_EDGES = 300000

NC = 2
NS = 16
NW = NC * NS

E_PAD = 307200
E1 = 163840
E2 = E_PAD - E1
CH = 128
KB = 10

FC_W = 16
N_FC = D_MSG // FC_W
ROWS_PER_SUB = 6272
ACC_ROWS = ROWS_PER_SUB * NS
LAST_ROWS = N_NODES - 15 * ROWS_PER_SUB


def _sc_mesh():
    return plsc.VectorSubcoreMesh(
        core_axis_name="c", subcore_axis_name="s", num_cores=NC,
        num_subcores=NS)


KG = 5


def _gather_body(nchw, mem_hbm, ts_hbm, src2_hbm, dst2_hbm,
                 smem_out, dmem_out, sts_out, dts_out,
                 idx_v, rows_v, ts_v, sem):
    wid = lax.axis_index("s") * NC + lax.axis_index("c")
    crow0 = wid * nchw

    def one_table(idx2_hbm, mem_out, tsg_out):
        @pl.loop(0, nchw // KG)
        def _(b):
            row0 = crow0 + b * KG
            e_off = row0 * CH
            pltpu.async_copy(idx2_hbm.at[pl.ds(row0, KG)], idx_v, sem).wait()
            descs = []
            for j in range(KG):
                descs.append(pltpu.async_copy(
                    mem_hbm.at[idx_v.at[j]],
                    rows_v.at[pl.ds(j * CH, CH)], sem))
                descs.append(pltpu.async_copy(
                    ts_hbm.at[idx_v.at[j]],
                    ts_v.at[pl.ds(j * CH, CH)], sem))
            for d in descs:
                d.wait()
            pltpu.sync_copy(rows_v, mem_out.at[pl.ds(e_off, KG * CH)])
            pltpu.sync_copy(ts_v, tsg_out.at[pl.ds(e_off, KG * CH)])

    one_table(src2_hbm, smem_out, sts_out)
    one_table(dst2_hbm, dmem_out, dts_out)


def _gather(memory, last_ts, src_h, dst_h):
    f32 = jnp.float32
    eh = src_h.shape[0]
    nchw = eh // (CH * NW)
    out_type = (
        jax.ShapeDtypeStruct((eh, D_MEM), f32),
        jax.ShapeDtypeStruct((eh, D_MEM), f32),
        jax.ShapeDtypeStruct((eh,), f32),
        jax.ShapeDtypeStruct((eh,), f32),
    )
    scratch = [
        pltpu.VMEM((KG, CH), jnp.int32),
        pltpu.VMEM((KG * CH, D_MEM), f32),
        pltpu.VMEM((KG * CH,), f32),
        pltpu.SemaphoreType.DMA,
    ]
    body = functools.partial(_gather_body, nchw)
    return pl.kernel(body, out_type=out_type, mesh=_sc_mesh(),
                     scratch_types=scratch,
                     compiler_params=pltpu.CompilerParams(
                         use_tc_tiling_on_sc=False))(
        memory, last_ts, src_h.reshape(-1, CH), dst_h.reshape(-1, CH))


BE = 2048


def _msg_body(off, smem, dmem, sts, dts, ts, ef, wa, wb, wc, wd, b, freq,
              phase, msrc_out, mdst_out):
    i = pl.program_id(0)
    row = jax.lax.broadcasted_iota(jnp.int32, (BE, 1), 0) + (i * BE + off)
    valid = row < N_EDGES

    s = smem[...]
    d = dmem[...]
    enc_s = jnp.cos((ts[...] - sts[...])[:, None] * freq[...][None, :]
                    + phase[...][None, :])
    enc_d = jnp.cos((ts[...] - dts[...])[:, None] * freq[...][None, :]
                    + phase[...][None, :])

    dot = functools.partial(jnp.dot, preferred_element_type=jnp.float32)
    s_a = dot(s, wa[...])
    s_b = dot(s, wb[...])
    d_a = dot(d, wa[...])
    d_b = dot(d, wb[...])
    e_d = dot(ef[...], wd[...])
    bias = b[...][None, :]
    mdst = jnp.maximum(s_a + d_b + dot(enc_d, wc[...]) + e_d + bias, 0.0)
    msrc = jnp.maximum(d_a + s_b + dot(enc_s, wc[...]) + e_d + bias, 0.0)
    msrc_out[...] = jnp.where(valid, msrc, 0.0)
    mdst_out[...] = jnp.where(valid, mdst, 0.0)


def _messages(off, smem, dmem, sts, dts, ts_h, ef_h, msg_W, msg_b,
              basis_freq, phase):
    f32 = jnp.float32
    eh = smem.shape[0]
    wa = msg_W[:, :D_MEM].T
    wb = msg_W[:, D_MEM:2 * D_MEM].T
    wc = msg_W[:, 2 * D_MEM:2 * D_MEM + D_TIME].T
    wd = msg_W[:, 2 * D_MEM + D_TIME:].T
    grid = (eh // BE,)
    eb = lambda i: (i, 0)
    vb = lambda i: (i,)
    full2 = lambda i: (0, 0)
    full1 = lambda i: (0,)
    in_specs = [
        pl.BlockSpec((BE, D_MEM), eb),
        pl.BlockSpec((BE, D_MEM), eb),
        pl.BlockSpec((BE,), vb),
        pl.BlockSpec((BE,), vb),
        pl.BlockSpec((BE,), vb),
        pl.BlockSpec((BE, D_EDGE), eb),
        pl.BlockSpec((D_MEM, D_MSG), full2),
        pl.BlockSpec((D_MEM, D_MSG), full2),
        pl.BlockSpec((D_TIME, D_MSG), full2),
        pl.BlockSpec((D_EDGE, D_MSG), full2),
        pl.BlockSpec((D_MSG,), full1),
        pl.BlockSpec((D_TIME,), full1),
        pl.BlockSpec((D_TIME,), full1),
    ]
    out_specs = [pl.BlockSpec((BE, D_MSG), eb), pl.BlockSpec((BE, D_MSG), eb)]
    out_shape = [jax.ShapeDtypeStruct((eh, D_MSG), f32),
                 jax.ShapeDtypeStruct((eh, D_MSG), f32)]
    body = functools.partial(_msg_body, off)
    return pl.pallas_call(body, grid=grid, in_specs=in_specs,
                          out_specs=out_specs, out_shape=out_shape)(
        smem, dmem, sts, dts, ts_h, ef_h, wa, wb, wc, wd, msg_b,
        basis_freq, phase)


NCH1 = E1 // NS // CH
NCH2 = E2 // NS // CH
VALID2 = N_EDGES - E1
FULL15 = (VALID2 - (NS - 1) * (E2 // NS)) // CH
MIXED15 = VALID2 - (NS - 1) * (E2 // NS) - FULL15 * CH


def _scatter_body(sidx1_hbm, didx1_hbm, sidx2_hbm, didx2_hbm,
                  msrc1_hbm, mdst1_hbm, msrc2_hbm, mdst2_hbm, zeros_hbm,
                  sums_out, cnta_out, cntb_out,
                  acc_sh, idxb_v, valb_v, ones_v, sem):
    c = lax.axis_index("c")
    s = lax.axis_index("s")

    def rows_split(fn):
        @pl.when(s < NS - 1)
        def _():
            fn(s * ROWS_PER_SUB, ROWS_PER_SUB)

        @pl.when(s == NS - 1)
        def _():
            fn((NS - 1) * ROWS_PER_SUB, LAST_ROWS)

    def zero_acc():
        def z(r0, nr):
            r0 = pl.multiple_of(r0, 8)
            pltpu.sync_copy(zeros_hbm.at[pl.ds(r0, nr)],
                            acc_sh.at[pl.ds(r0, nr)])
        rows_split(z)

    def scan_array(idx2_hbm, val_hbm, fc, nchunk):
        crow = s * nchunk

        @pl.loop(0, nchunk // KB)
        def _(b):
            row0 = crow + b * KB
            e_off = row0 * CH
            cp1 = pltpu.async_copy(idx2_hbm.at[pl.ds(row0, KB)], idxb_v, sem)
            cp2 = pltpu.async_copy(
                val_hbm.at[pl.ds(e_off, KB * CH), pl.ds(fc * FC_W, FC_W)],
                valb_v, sem)
            cp1.wait()
            cp2.wait()
            descs = []
            for j in range(KB):
                descs.append(pltpu.async_copy(
                    valb_v.at[pl.ds(j * CH, CH)],
                    acc_sh.at[idxb_v.at[j]], sem, add=True))
            for d in descs:
                d.wait()

    def drain(out_ref, fc):
        def d(r0, nr):
            r0 = pl.multiple_of(r0, 8)
            pltpu.sync_copy(
                acc_sh.at[pl.ds(r0, nr)],
                out_ref.at[pl.ds(r0, nr), pl.ds(fc * FC_W, FC_W)])
        rows_split(d)

    for fc in range(N_FC):
        @pl.when(c == fc // (N_FC // NC))
        def _(fc=fc):
            zero_acc()
            plsc.subcore_barrier()
            scan_array(sidx1_hbm, msrc1_hbm, fc, NCH1)
            scan_array(didx1_hbm, mdst1_hbm, fc, NCH1)
            scan_array(sidx2_hbm, msrc2_hbm, fc, NCH2)
            scan_array(didx2_hbm, mdst2_hbm, fc, NCH2)
            plsc.subcore_barrier()
            drain(sums_out, fc)
            plsc.subcore_barrier()

    def fill_ones(valid):
        @pl.loop(0, CH)
        def _(r):
            ones_v[r, :] = (jnp.where(r < valid, 1.0, 0.0)
                            * jnp.ones((FC_W,), jnp.float32))

    def count_scan(idx1_hbm, idx2_hbm):
        def count_batch(idx2, row0, n):
            pltpu.async_copy(idx2.at[pl.ds(row0, n)],
                             idxb_v.at[pl.ds(0, n)], sem).wait()
            descs = [pltpu.async_copy(ones_v, acc_sh.at[idxb_v.at[j]],
                                      sem, add=True) for j in range(n)]
            for d in descs:
                d.wait()

        crow1 = s * NCH1
        @pl.loop(0, NCH1 // KB)
        def _(i):
            count_batch(idx1_hbm, crow1 + i * KB, KB)

        crow2 = s * NCH2

        @pl.when(s < NS - 1)
        def _():
            @pl.loop(0, NCH2 // KB)
            def _(i):
                count_batch(idx2_hbm, crow2 + i * KB, KB)

        @pl.when(s == NS - 1)
        def _():
            nfull_b = FULL15 // KB
            rem = FULL15 - nfull_b * KB

            @pl.loop(0, nfull_b)
            def _(i):
                count_batch(idx2_hbm, crow2 + i * KB, KB)
            count_batch(idx2_hbm, crow2 + nfull_b * KB, rem)
            fill_ones(MIXED15)
            count_batch(idx2_hbm, crow2 + FULL15, 1)

    for side in range(NC):
        @pl.when(c == side)
        def _(side=side):
            zero_acc()
            fill_ones(CH)
            plsc.subcore_barrier()
            count_scan((sidx1_hbm, didx1_hbm)[side],
                       (sidx2_hbm, didx2_hbm)[side])
            plsc.subcore_barrier()
            out_ref = (cnta_out, cntb_out)[side]

            def dc(r0, nr):
                r0 = pl.multiple_of(r0, 8)
                pltpu.sync_copy(acc_sh.at[pl.ds(r0, nr)],
                                out_ref.at[pl.ds(r0, nr), pl.ds(0, FC_W)])
            rows_split(dc)


def _scatter(src1, dst1, src2, dst2, msrc1, mdst1, msrc2, mdst2, zeros2d):
    f32 = jnp.float32
    out_type = (
        jax.ShapeDtypeStruct((N_NODES, D_MSG), f32),
        jax.ShapeDtypeStruct((ACC_ROWS, 128), f32),
        jax.ShapeDtypeStruct((ACC_ROWS, 128), f32),
    )
    scratch = [
        pltpu.VMEM_SHARED((ACC_ROWS, FC_W), f32),
        pltpu.VMEM((KB, CH), jnp.int32),
        pltpu.VMEM((KB * CH, FC_W), f32),
        pltpu.VMEM((CH, FC_W), f32),
        pltpu.SemaphoreType.DMA,
    ]
    return pl.kernel(_scatter_body, out_type=out_type, mesh=_sc_mesh(),
                     scratch_types=scratch,
                     compiler_params=pltpu.CompilerParams(
                         use_tc_tiling_on_sc=False))(
        src1.reshape(-1, CH), dst1.reshape(-1, CH),
        src2.reshape(-1, CH), dst2.reshape(-1, CH),
        msrc1, mdst1, msrc2, mdst2, zeros2d)


NB = 2000


def _gru_body(sums, cnta, cntb, mem, wih, whh, bih, bhh, out):
    cnt = cnta[...][:, :1] + cntb[...][:, :1]
    agg = sums[...] / jnp.maximum(cnt, 1.0)
    h = mem[...]
    gi = jnp.dot(agg, wih[...], preferred_element_type=jnp.float32) \
        + bih[...][None, :]
    gh = jnp.dot(h, whh[...], preferred_element_type=jnp.float32) \
        + bhh[...][None, :]
    r = jax.nn.sigmoid(gi[:, :D_MEM] + gh[:, :D_MEM])
    z = jax.nn.sigmoid(gi[:, D_MEM:2 * D_MEM] + gh[:, D_MEM:2 * D_MEM])
    n = jnp.tanh(gi[:, 2 * D_MEM:] + r * gh[:, 2 * D_MEM:])
    hn = (1.0 - z) * n + z * h
    out[...] = jnp.where(cnt > 0.0, hn, h)


def _gru(sums, cnta, cntb, memory, gru_W_ih, gru_W_hh, gru_b_ih, gru_b_hh):
    f32 = jnp.float32
    wih = gru_W_ih.T
    whh = gru_W_hh.T
    grid = (N_NODES // NB,)
    nb = lambda i: (i, 0)
    full2 = lambda i: (0, 0)
    full1 = lambda i: (0,)
    in_specs = [
        pl.BlockSpec((NB, D_MSG), nb),
        pl.BlockSpec((NB, 128), nb),
        pl.BlockSpec((NB, 128), nb),
        pl.BlockSpec((NB, D_MEM), nb),
        pl.BlockSpec((D_MSG, 3 * D_MEM), full2),
        pl.BlockSpec((D_MEM, 3 * D_MEM), full2),
        pl.BlockSpec((3 * D_MEM,), full1),
        pl.BlockSpec((3 * D_MEM,), full1),
    ]
    out_specs = pl.BlockSpec((NB, D_MEM), nb)
    out_shape = jax.ShapeDtypeStruct((N_NODES, D_MEM), f32)
    return pl.pallas_call(_gru_body, grid=grid, in_specs=in_specs,
                          out_specs=out_specs, out_shape=out_shape)(
        sums, cnta, cntb, memory, wih, whh, gru_b_ih, gru_b_hh)


def kernel(src_nodes, dst_nodes, timestamps, edge_features, memory,
           last_update_ts, basis_freq, phase, msg_W, msg_b,
           gru_W_ih, gru_W_hh, gru_b_ih, gru_b_hh):
    f32 = jnp.float32
    npad = E_PAD - N_EDGES
    pad_idx = (jnp.arange(npad, dtype=jnp.int32) * 1009) % N_NODES
    src_pad = jnp.concatenate([src_nodes.astype(jnp.int32), pad_idx])
    dst_pad = jnp.concatenate([dst_nodes.astype(jnp.int32), pad_idx])
    ts_pad = jnp.concatenate([timestamps, jnp.zeros((npad,), f32)])
    ef_pad = jnp.concatenate(
        [edge_features, jnp.zeros((npad, D_EDGE), f32)], axis=0)
    zeros2d = jnp.zeros((N_NODES, FC_W), f32)

    src1, src2 = src_pad[:E1], src_pad[E1:]
    dst1, dst2 = dst_pad[:E1], dst_pad[E1:]

    s1m, d1m, s1t, d1t = _gather(memory, last_update_ts, src1, dst1)
    s2m, d2m, s2t, d2t = _gather(memory, last_update_ts, src2, dst2)
    msrc1, mdst1 = _messages(0, s1m, d1m, s1t, d1t, ts_pad[:E1], ef_pad[:E1],
                             msg_W, msg_b, basis_freq, phase)
    msrc2, mdst2 = _messages(E1, s2m, d2m, s2t, d2t, ts_pad[E1:], ef_pad[E1:],
                             msg_W, msg_b, basis_freq, phase)
    sums, cnta, cntb = _scatter(src1, dst1, src2, dst2,
                                msrc1, mdst1, msrc2, mdst2, zeros2d)
    return _gru(sums, cnta, cntb, memory, gru_W_ih, gru_W_hh,
                gru_b_ih, gru_b_hh)
